# baseline (speedup 1.0000x reference)
import jax
import jax.numpy as jnp
from jax.experimental import pallas as pl
from jax.experimental.pallas import tpu as pltpu


def _gcn_kernel(adj_ref, x_ref, w_ref, b_ref, out_ref, s_ref):
    @pl.when(pl.program_id(0) == 0)
    def _():
        s_ref[...] = jnp.dot(
            x_ref[...], w_ref[...], preferred_element_type=jnp.float32
        ).astype(jnp.bfloat16)

    acc = jnp.dot(
        adj_ref[...].astype(jnp.bfloat16),
        s_ref[...],
        preferred_element_type=jnp.float32,
    )
    out_ref[...] = jnp.maximum(acc + b_ref[...], 0.0)


def kernel(x, adj, W, b):
    n, nfeat = x.shape
    nhid = W.shape[1]

    bm = 200
    out = pl.pallas_call(
        _gcn_kernel,
        grid=(n // bm,),
        in_specs=[
            pl.BlockSpec((bm, n), lambda i: (i, 0)),
            pl.BlockSpec((n, nfeat), lambda i: (0, 0)),
            pl.BlockSpec((nfeat, nhid), lambda i: (0, 0)),
            pl.BlockSpec((1, nhid), lambda i: (0, 0)),
        ],
        out_specs=pl.BlockSpec((bm, nhid), lambda i: (i, 0)),
        out_shape=jax.ShapeDtypeStruct((n, nhid), jnp.float32),
        scratch_shapes=[pltpu.VMEM((n, nhid), jnp.bfloat16)],
    )(adj, x, W, b.reshape(1, nhid))
    return out


# speedup vs baseline: 1.0055x; 1.0055x over previous
import jax
import jax.numpy as jnp
from jax.experimental import pallas as pl
from jax.experimental.pallas import tpu as pltpu


def _gcn_kernel(adj_ref, x_ref, w_ref, b_ref, out_ref, s_ref):
    @pl.when(pl.program_id(0) == 0)
    def _():
        s_ref[...] = jnp.dot(
            x_ref[...], w_ref[...], preferred_element_type=jnp.float32
        )

    acc = jnp.dot(
        adj_ref[...],
        s_ref[...],
        preferred_element_type=jnp.float32,
    )
    out_ref[...] = jnp.maximum(acc + b_ref[...], 0.0)


def kernel(x, adj, W, b):
    n, nfeat = x.shape
    nhid = W.shape[1]

    bm = 200
    out = pl.pallas_call(
        _gcn_kernel,
        grid=(n // bm,),
        in_specs=[
            pl.BlockSpec((bm, n), lambda i: (i, 0)),
            pl.BlockSpec((n, nfeat), lambda i: (0, 0)),
            pl.BlockSpec((nfeat, nhid), lambda i: (0, 0)),
            pl.BlockSpec((1, nhid), lambda i: (0, 0)),
        ],
        out_specs=pl.BlockSpec((bm, nhid), lambda i: (i, 0)),
        out_shape=jax.ShapeDtypeStruct((n, nhid), jnp.float32),
        scratch_shapes=[pltpu.VMEM((n, nhid), jnp.float32)],
    )(adj, x, W, b.reshape(1, nhid))
    return out


# speedup vs baseline: 1.0183x; 1.0128x over previous
import jax
import jax.numpy as jnp
from jax.experimental import pallas as pl
from jax.experimental.pallas import tpu as pltpu


def _gcn_kernel(adj_ref, x_ref, w_ref, b_ref, out_ref, s_ref):
    @pl.when(pl.program_id(0) == 0)
    def _():
        s_ref[...] = jnp.dot(
            x_ref[...], w_ref[...], preferred_element_type=jnp.float32
        )

    acc = jnp.dot(
        adj_ref[...],
        s_ref[...],
        preferred_element_type=jnp.float32,
    )
    out_ref[...] = jnp.maximum(acc + b_ref[...], 0.0)


def kernel(x, adj, W, b):
    n, nfeat = x.shape
    nhid = W.shape[1]

    bm = 400
    out = pl.pallas_call(
        _gcn_kernel,
        grid=(n // bm,),
        in_specs=[
            pl.BlockSpec((bm, n), lambda i: (i, 0)),
            pl.BlockSpec((n, nfeat), lambda i: (0, 0)),
            pl.BlockSpec((nfeat, nhid), lambda i: (0, 0)),
            pl.BlockSpec((1, nhid), lambda i: (0, 0)),
        ],
        out_specs=pl.BlockSpec((bm, nhid), lambda i: (i, 0)),
        out_shape=jax.ShapeDtypeStruct((n, nhid), jnp.float32),
        scratch_shapes=[pltpu.VMEM((n, nhid), jnp.float32)],
    )(adj, x, W, b.reshape(1, nhid))
    return out
